# baseline (speedup 1.0000x reference)
import functools

import jax
import jax.numpy as jnp
from jax import lax
from jax.experimental import pallas as pl
from jax.experimental.pallas import tpu as pltpu
from jax.experimental.pallas import tpu_sc as plsc

NC = 2
NS = 16
L = 16

B = 4
T = 4096
D = 2048
K = T // 2
RADIX = 256
PASSES = 4
CHUNK = T // L


def _digit(k_i32, shift):
    ku = plsc.bitcast(k_i32, jnp.uint32)
    du = jnp.bitwise_and(jnp.right_shift(ku, jnp.uint32(shift)), jnp.uint32(RADIX - 1))
    return plsc.bitcast(du, jnp.int32)


def _desc_key(bits_i32):
    sign = jnp.right_shift(bits_i32, 31)
    mask = jnp.bitwise_and(jnp.bitwise_not(sign), jnp.int32(0x7FFFFFFF))
    return jnp.bitwise_xor(bits_i32, mask)


def _topk_body(scores_hbm, vals_hbm, idx_hbm, bidx_hbm, grow_hbm,
               sc_v, key_a, key_b, val_a, val_b, hist, offs,
               vstage, gstage, bstage):
    wid = lax.axis_index("s") * NC + lax.axis_index("c")

    @pl.when(wid < B)
    def _():
        b = wid
        pltpu.sync_copy(scores_hbm.at[pl.ds(b * T, T)], sc_v)

        lane = lax.iota(jnp.int32, L)
        lane_c = lane * CHUNK
        ones = jnp.broadcast_to(jnp.int32(1), (L,))

        def init_body(i, _):
            x = sc_v[pl.ds(i * L, L)]
            bits = plsc.bitcast(x, jnp.int32)
            key_a[pl.ds(i * L, L)] = _desc_key(bits)
            val_a[pl.ds(i * L, L)] = lane + i * L
            return 0
        lax.fori_loop(0, T // L, init_body, 0)

        bufs = [(key_a, val_a), (key_b, val_b)]
        for p in range(PASSES):
            shift = 8 * p
            src_k, src_v = bufs[p % 2]
            dst_k, dst_v = bufs[(p + 1) % 2]

            def zero_body(j, _):
                hist[j, :] = jnp.broadcast_to(jnp.int32(0), (L,))
                return 0
            lax.fori_loop(0, RADIX, zero_body, 0)

            def hist_body(i, _):
                idxv = lane_c + i
                k = plsc.load_gather(src_k, [idxv])
                d = _digit(k, shift)
                plsc.addupdate_scatter(hist, [d, lane], ones)
                return 0
            lax.fori_loop(0, CHUNK, hist_body, 0)

            def offs_body(dd, carry):
                row = hist[dd, :]
                cs = plsc.cumsum(row)
                offs[dd, :] = cs - row + carry
                return carry + jnp.sum(row)
            lax.fori_loop(0, RADIX, offs_body, jnp.int32(0))

            def perm_body(i, _):
                idxv = lane_c + i
                k = plsc.load_gather(src_k, [idxv])
                v = plsc.load_gather(src_v, [idxv])
                d = _digit(k, shift)
                ofs = plsc.load_gather(offs, [d, lane])
                plsc.store_scatter(dst_k, [ofs], k)
                plsc.store_scatter(dst_v, [ofs], v)
                plsc.addupdate_scatter(offs, [d, lane], ones)
                return 0
            lax.fori_loop(0, CHUNK, perm_body, 0)

        def out_body(i, _):
            k = key_a[pl.ds(i * L, L)]
            v = val_a[pl.ds(i * L, L)]
            vstage[pl.ds(i * L, L)] = plsc.bitcast(_desc_key(k), jnp.float32)
            gstage[pl.ds(i * L, L)] = v + b * T
            bstage[pl.ds(i * L, L)] = jnp.broadcast_to(b, (L,))
            return 0
        lax.fori_loop(0, K // L, out_body, 0)

        pltpu.sync_copy(vstage, vals_hbm.at[pl.ds(b * K, K)])
        pltpu.sync_copy(val_a.at[pl.ds(0, K)], idx_hbm.at[pl.ds(b * K, K)])
        pltpu.sync_copy(gstage, grow_hbm.at[pl.ds(b * K, K)])
        pltpu.sync_copy(bstage, bidx_hbm.at[pl.ds(b * K, K)])


_topk_call = functools.partial(
    pl.kernel,
    out_type=(
        jax.ShapeDtypeStruct((B * K,), jnp.float32),
        jax.ShapeDtypeStruct((B * K,), jnp.int32),
        jax.ShapeDtypeStruct((B * K,), jnp.int32),
        jax.ShapeDtypeStruct((B * K,), jnp.int32),
    ),
    mesh=plsc.VectorSubcoreMesh(core_axis_name="c", subcore_axis_name="s"),
    compiler_params=pltpu.CompilerParams(needs_layout_passes=False),
    scratch_types=[
        pltpu.VMEM((T,), jnp.float32),
        pltpu.VMEM((T,), jnp.int32),
        pltpu.VMEM((T,), jnp.int32),
        pltpu.VMEM((T,), jnp.int32),
        pltpu.VMEM((T,), jnp.int32),
        pltpu.VMEM((RADIX, L), jnp.int32),
        pltpu.VMEM((RADIX, L), jnp.int32),
        pltpu.VMEM((K,), jnp.float32),
        pltpu.VMEM((K,), jnp.int32),
        pltpu.VMEM((K,), jnp.int32),
    ],
)(_topk_body)


NW = NC * NS
RPW = (B * K) // NW
GCH = 16
NCH = RPW // GCH


def _gather_body(hid_hbm, grow_hbm, out_hbm, idx_v,
                 buf0, buf1, buf2, gs0, gs1, gs2, os0, os1, os2):
    wid = lax.axis_index("s") * NC + lax.axis_index("c")
    base = wid * RPW
    pltpu.sync_copy(grow_hbm.at[pl.ds(base, RPW)], idx_v)

    bufs = (buf0, buf1, buf2)
    gsems = (gs0, gs1, gs2)
    osems = (os0, os1, os2)

    def start_g(c):
        return pltpu.async_copy(
            hid_hbm.at[idx_v.at[pl.ds(c * GCH, GCH)]], bufs[c % 3], gsems[c % 3])

    def start_o(c):
        return pltpu.async_copy(
            bufs[c % 3], out_hbm.at[pl.ds(base + c * GCH, GCH)], osems[c % 3])

    hg = [None] * NCH
    ho = [None] * NCH
    hg[0] = start_g(0)
    hg[1] = start_g(1)
    for c in range(NCH):
        hg[c].wait()
        ho[c] = start_o(c)
        if c + 2 < NCH:
            if c - 1 >= 0:
                ho[c - 1].wait()
            hg[c + 2] = start_g(c + 2)
    ho[NCH - 2].wait()
    ho[NCH - 1].wait()


_gather_call = functools.partial(
    pl.kernel,
    out_type=jax.ShapeDtypeStruct((B * K, D), jnp.float32),
    mesh=plsc.VectorSubcoreMesh(core_axis_name="c", subcore_axis_name="s"),
    scratch_types=[
        pltpu.VMEM((RPW,), jnp.int32),
        pltpu.VMEM((GCH, D), jnp.float32),
        pltpu.VMEM((GCH, D), jnp.float32),
        pltpu.VMEM((GCH, D), jnp.float32),
        pltpu.SemaphoreType.DMA,
        pltpu.SemaphoreType.DMA,
        pltpu.SemaphoreType.DMA,
        pltpu.SemaphoreType.DMA,
        pltpu.SemaphoreType.DMA,
        pltpu.SemaphoreType.DMA,
    ],
)(_gather_body)


def kernel(scores, hidden_states):
    b, t, d = hidden_states.shape
    vals, idx, bidx, grow = _topk_call(scores.reshape(-1))
    sel = _gather_call(hidden_states.reshape(b * t, d), grow)
    return sel, bidx, idx, vals


# speedup vs baseline: 1.0283x; 1.0283x over previous
import functools

import jax
import jax.numpy as jnp
from jax import lax
from jax.experimental import pallas as pl
from jax.experimental.pallas import tpu as pltpu
from jax.experimental.pallas import tpu_sc as plsc

NC = 2
NS = 16
L = 16

B = 4
T = 4096
D = 2048
K = T // 2
RADIX = 256
PASSES = 4
CHUNK = T // L

RPC = 2 * K
RPW = RPC // NS
GCH = 8
NCH = RPW // GCH


def _digit(k_i32, shift):
    ku = plsc.bitcast(k_i32, jnp.uint32)
    du = jnp.bitwise_and(jnp.right_shift(ku, jnp.uint32(shift)), jnp.uint32(RADIX - 1))
    return plsc.bitcast(du, jnp.int32)


def _desc_key(bits_i32):
    sign = jnp.right_shift(bits_i32, 31)
    mask = jnp.bitwise_and(jnp.bitwise_not(sign), jnp.int32(0x7FFFFFFF))
    return jnp.bitwise_xor(bits_i32, mask)


def _body(scores_hbm, hid_hbm, sel_hbm, bidx_hbm, idx_hbm, vals_hbm,
          sc_v, key_a, key_b, val_a, val_b, hist, offs,
          gstage, idx_v, buf0, buf1, sh_grow, sem0, sem1):
    c = lax.axis_index("c")
    s = lax.axis_index("s")

    @pl.when(s < 2)
    def _():
        b = c * 2 + s
        pltpu.sync_copy(scores_hbm.at[pl.ds(b * T, T)], sc_v)

        lane = lax.iota(jnp.int32, L)
        lane_c = lane * CHUNK
        ones = jnp.broadcast_to(jnp.int32(1), (L,))

        def init_body(i, _):
            x = sc_v[pl.ds(i * L, L)]
            bits = plsc.bitcast(x, jnp.int32)
            key_a[pl.ds(i * L, L)] = _desc_key(bits)
            val_a[pl.ds(i * L, L)] = lane + i * L
            return 0
        lax.fori_loop(0, T // L, init_body, 0)

        bufs = [(key_a, val_a), (key_b, val_b)]
        for p in range(PASSES):
            shift = 8 * p
            src_k, src_v = bufs[p % 2]
            dst_k, dst_v = bufs[(p + 1) % 2]

            def zero_body(j, _):
                hist[j, :] = jnp.broadcast_to(jnp.int32(0), (L,))
                return 0
            lax.fori_loop(0, RADIX, zero_body, 0)

            def hist_body(i, _):
                idxv = lane_c + i
                k = plsc.load_gather(src_k, [idxv])
                d = _digit(k, shift)
                plsc.addupdate_scatter(hist, [d, lane], ones)
                return 0
            lax.fori_loop(0, CHUNK, hist_body, 0)

            def offs_body(dd, carry):
                row = hist[dd, :]
                cs = plsc.cumsum(row)
                offs[dd, :] = cs - row + carry
                return carry + jnp.sum(row)
            lax.fori_loop(0, RADIX, offs_body, jnp.int32(0))

            def perm_body(i, _):
                idxv = lane_c + i
                k = plsc.load_gather(src_k, [idxv])
                v = plsc.load_gather(src_v, [idxv])
                d = _digit(k, shift)
                ofs = plsc.load_gather(offs, [d, lane])
                plsc.store_scatter(dst_k, [ofs], k)
                plsc.store_scatter(dst_v, [ofs], v)
                plsc.addupdate_scatter(offs, [d, lane], ones)
                return 0
            lax.fori_loop(0, CHUNK, perm_body, 0)

        def out_body(i, _):
            k = key_a[pl.ds(i * L, L)]
            v = val_a[pl.ds(i * L, L)]
            sc_v[pl.ds(i * L, L)] = plsc.bitcast(_desc_key(k), jnp.float32)
            gstage[pl.ds(i * L, L)] = v + b * T
            return 0
        lax.fori_loop(0, K // L, out_body, 0)

        pltpu.sync_copy(sc_v.at[pl.ds(0, K)], vals_hbm.at[pl.ds(b * K, K)])
        pltpu.sync_copy(val_a.at[pl.ds(0, K)], idx_hbm.at[pl.ds(b * K, K)])
        pltpu.sync_copy(gstage, sh_grow.at[pl.ds(s * K, K)])

        def bidx_body(i, _):
            gstage[pl.ds(i * L, L)] = jnp.broadcast_to(b, (L,))
            return 0
        lax.fori_loop(0, K // L, bidx_body, 0)
        pltpu.sync_copy(gstage, bidx_hbm.at[pl.ds(b * K, K)])

    plsc.subcore_barrier()

    pltpu.sync_copy(sh_grow.at[pl.ds(s * RPW, RPW)], idx_v)
    gbase = c * RPC + s * RPW

    bufs2 = (buf0, buf1)
    sems2 = (sem0, sem1)

    def start(ch):
        return pltpu.async_copy(
            hid_hbm.at[idx_v.at[pl.ds(ch * GCH, GCH)]], bufs2[ch % 2], sems2[ch % 2])

    pending = start(0)
    for ch in range(NCH):
        nxt = start(ch + 1) if ch + 1 < NCH else None
        pending.wait()
        pltpu.sync_copy(bufs2[ch % 2], sel_hbm.at[pl.ds(gbase + ch * GCH, GCH)])
        pending = nxt


_fused_call = functools.partial(
    pl.kernel,
    out_type=(
        jax.ShapeDtypeStruct((B * K, D), jnp.float32),
        jax.ShapeDtypeStruct((B * K,), jnp.int32),
        jax.ShapeDtypeStruct((B * K,), jnp.int32),
        jax.ShapeDtypeStruct((B * K,), jnp.float32),
    ),
    mesh=plsc.VectorSubcoreMesh(core_axis_name="c", subcore_axis_name="s"),
    compiler_params=pltpu.CompilerParams(needs_layout_passes=False),
    scratch_types=[
        pltpu.VMEM((T,), jnp.float32),
        pltpu.VMEM((T,), jnp.int32),
        pltpu.VMEM((T,), jnp.int32),
        pltpu.VMEM((T,), jnp.int32),
        pltpu.VMEM((T,), jnp.int32),
        pltpu.VMEM((RADIX, L), jnp.int32),
        pltpu.VMEM((RADIX, L), jnp.int32),
        pltpu.VMEM((K,), jnp.int32),
        pltpu.VMEM((RPW,), jnp.int32),
        pltpu.VMEM((GCH, D), jnp.float32),
        pltpu.VMEM((GCH, D), jnp.float32),
        pltpu.VMEM_SHARED((RPC,), jnp.int32),
        pltpu.SemaphoreType.DMA,
        pltpu.SemaphoreType.DMA,
    ],
)(_body)


def kernel(scores, hidden_states):
    b, t, d = hidden_states.shape
    sel, bidx, idx, vals = _fused_call(
        scores.reshape(-1), hidden_states.reshape(b * t, d))
    return sel, bidx, idx, vals
